# baseline (speedup 1.0000x reference)
import functools

import jax
import jax.numpy as jnp
from jax import lax
from jax.experimental import pallas as pl
from jax.experimental.pallas import tpu as pltpu
from jax.experimental.pallas import tpu_sc as plsc

N = 10000
E = 640000
LSEQ = 20
VOCAB = 100000
IP = 128
FP = 128
HG = 128
NP = 10112
BN = 632
NBLK = NP // BN
BN7 = 1000

NC = 2
NS = 16
NW = NC * NS

MP_NPASS = 3
P_BLK = 80
P_STG = 10752
P_CAP = 21504
P_DUMP = 10752
P_TAIL = 512
MP_RANGE = 3392
ACC_ROWS = 3456
JUNK = ACC_ROWS - 1
NPOUT = MP_NPASS * MP_RANGE
DEG_R = 10240

MP_CS = 128
MP_CH = 160
EP = NW * MP_CH * MP_CS
MP_ROWS = EP // MP_CS

EMB_CS = 80
EMB_CH = 79
EMB_PER_W = EMB_CH * EMB_CS

_MESH = plsc.VectorSubcoreMesh(core_axis_name="c", subcore_axis_name="s",
                               num_cores=NC, num_subcores=NS)
_SC_PARAMS = pltpu.CompilerParams(needs_layout_passes=False)


def _wid():
    return lax.axis_index("c") * NS + lax.axis_index("s")


def _emb_body(idx_hbm, table_hbm, out_hbm, idx_v, rows_a, rows_b, sem_a, sem_b):
    w = _wid()
    base = w * EMB_PER_W
    pltpu.sync_copy(idx_hbm.at[pl.ds(base, EMB_PER_W)], idx_v)

    def idx_at(k):
        return idx_v.at[pl.ds(k * EMB_CS, EMB_CS)]

    def gather(k, buf, sem):
        return pltpu.async_copy(table_hbm.at[idx_at(k)], buf, sem)

    def put(k, buf):
        pltpu.sync_copy(buf, out_hbm.at[pl.ds(base + k * EMB_CS, EMB_CS), :])

    gather(0, rows_a, sem_a)

    def pair(j, carry):
        gather(2 * j + 1, rows_b, sem_b)
        pltpu.make_async_copy(table_hbm.at[idx_at(2 * j)], rows_a, sem_a).wait()
        put(2 * j, rows_a)
        gather(2 * j + 2, rows_a, sem_a)
        pltpu.make_async_copy(table_hbm.at[idx_at(2 * j + 1)], rows_b, sem_b).wait()
        put(2 * j + 1, rows_b)
        return carry

    lax.fori_loop(0, (EMB_CH - 1) // 2, pair, 0)
    k_last = EMB_CH - 1
    pltpu.make_async_copy(table_hbm.at[idx_at(k_last)], rows_a, sem_a).wait()
    put(k_last, rows_a)


_emb_kernel = functools.partial(
    pl.kernel,
    out_type=jax.ShapeDtypeStruct((LSEQ * NP, IP), jnp.float32),
    mesh=_MESH,
    compiler_params=_SC_PARAMS,
    scratch_types=[
        pltpu.VMEM((EMB_PER_W,), jnp.int32),
        pltpu.VMEM((EMB_CS, IP), jnp.float32),
        pltpu.VMEM((EMB_CS, IP), jnp.float32),
        pltpu.SemaphoreType.DMA,
        pltpu.SemaphoreType.DMA,
    ],
)(_emb_body)


def _part_body(src2_hbm, dst2_hbm, psrc_hbm, pdst_hbm, cnt_hbm,
               sblk_v, dblk_v, st_s0, st_s1, st_s2, st_d0, st_d1, st_d2, cnt_v):
    c = lax.axis_index("c")
    s = lax.axis_index("s")
    w = c * NS + s
    st_s = (st_s0, st_s1, st_s2)
    st_d = (st_d0, st_d1, st_d2)
    goff = [jnp.int32(0)] * MP_NPASS

    for blk in range(2):
        pltpu.sync_copy(src2_hbm.at[pl.ds(w * MP_CH + blk * P_BLK, P_BLK)], sblk_v)
        pltpu.sync_copy(dst2_hbm.at[pl.ds(w * MP_CH + blk * P_BLK, P_BLK)], dblk_v)

        def sub(t, offs):
            k = t // (MP_CS // 16)
            i = t % (MP_CS // 16)
            vs = sblk_v[k, pl.ds(i * 16, 16)]
            vd = dblk_v[k, pl.ds(i * 16, 16)]
            new = []
            for p in range(MP_NPASS):
                loc = vd - p * MP_RANGE
                ok = (loc >= 0) & (loc < MP_RANGE)
                off = offs[p]
                ranks = plsc.cumsum(ok.astype(jnp.int32))
                pos = off + ranks - 1
                plsc.store_scatter(st_s[p], [pos], vs, mask=ok)
                plsc.store_scatter(st_d[p], [pos], loc, mask=ok)
                new.append(off + jnp.sum(ok.astype(jnp.int32)))
            return tuple(new)

        offs = lax.fori_loop(0, P_BLK * (MP_CS // 16), sub, (jnp.int32(0),) * MP_NPASS)

        lanes16 = lax.iota(jnp.int32, 16)
        zeros16v = jnp.zeros((16,), jnp.int32)
        junk16v = jnp.full((16,), JUNK, jnp.int32)
        for p in range(MP_NPASS):
            off = offs[p]
            if blk == 0:
                plsc.store_scatter(st_s[p], [off + lanes16], zeros16v)
                plsc.store_scatter(st_d[p], [off + lanes16], junk16v)
                off = (off + 15) & ~jnp.int32(15)
            else:
                for q in range(32):
                    plsc.store_scatter(st_s[p], [off + q * 16 + lanes16], zeros16v)
                    plsc.store_scatter(st_d[p], [off + q * 16 + lanes16], junk16v)
            base = pl.multiple_of((w * MP_NPASS + p) * P_CAP + goff[p], 16)
            pltpu.sync_copy(st_s[p].at[pl.ds(0, P_DUMP)], psrc_hbm.at[pl.ds(base, P_DUMP)])
            pltpu.sync_copy(st_d[p].at[pl.ds(0, P_DUMP)], pdst_hbm.at[pl.ds(base, P_DUMP)])
            goff[p] = goff[p] + off

    np0 = (jnp.maximum(goff[0], 1) + 511) // 512
    np1 = (jnp.maximum(goff[1], 1) + 511) // 512
    np2 = (jnp.maximum(goff[2], 1) + 511) // 512
    lanes = lax.iota(jnp.int32, 16)
    cnt_v[...] = (jnp.where(lanes == 0, np0, 0) + jnp.where(lanes == 1, np1, 0)
                  + jnp.where(lanes == 2, np2, 0))
    pltpu.sync_copy(cnt_v, cnt_hbm.at[w])


_part_kernel = functools.partial(
    pl.kernel,
    out_type=[
        jax.ShapeDtypeStruct((NW * MP_NPASS * P_CAP + P_TAIL,), jnp.int32),
        jax.ShapeDtypeStruct((NW * MP_NPASS * P_CAP + P_TAIL,), jnp.int32),
        jax.ShapeDtypeStruct((NW, 16), jnp.int32),
    ],
    mesh=_MESH,
    compiler_params=_SC_PARAMS,
    scratch_types=[
        pltpu.VMEM((P_BLK, MP_CS), jnp.int32),
        pltpu.VMEM((P_BLK, MP_CS), jnp.int32),
        pltpu.VMEM((P_STG,), jnp.int32),
        pltpu.VMEM((P_STG,), jnp.int32),
        pltpu.VMEM((P_STG,), jnp.int32),
        pltpu.VMEM((P_STG,), jnp.int32),
        pltpu.VMEM((P_STG,), jnp.int32),
        pltpu.VMEM((P_STG,), jnp.int32),
        pltpu.VMEM((16,), jnp.int32),
    ],
)(_part_body)


def _deg_body(dst2_hbm, out_hbm, idx_v, hist_v, buf_v, res_v, sh):
    c = lax.axis_index("c")
    s = lax.axis_index("s")
    w = c * NS + s
    pltpu.sync_copy(dst2_hbm.at[pl.ds(w * MP_CH, MP_CH)], idx_v)

    def zero(i, carry):
        hist_v[pl.ds(i * 16, 16)] = jnp.zeros((16,), jnp.float32)
        return carry

    lax.fori_loop(0, DEG_R // 16, zero, 0)

    def count(k, carry):
        for i in range(MP_CS // 16):
            d = idx_v[k, pl.ds(i * 16, 16)]
            cnt, last = plsc.scan_count(d)
            plsc.addupdate_scatter(hist_v, [d], cnt.astype(jnp.float32),
                                   mask=last)
        return carry

    lax.fori_loop(0, MP_CH, count, 0)

    pltpu.sync_copy(hist_v, sh.at[s])
    plsc.subcore_barrier()
    for r in range(NS):
        pltpu.sync_copy(sh.at[r, pl.ds(s * (DEG_R // NS), DEG_R // NS)],
                        buf_v.at[r])

    def red(j, carry):
        tot = buf_v[0, pl.ds(j * 16, 16)]
        for r in range(1, NS):
            tot = tot + buf_v[r, pl.ds(j * 16, 16)]
        res_v[pl.ds(j * 16, 16)] = tot
        return carry

    lax.fori_loop(0, DEG_R // NS // 16, red, 0)
    pltpu.sync_copy(res_v, out_hbm.at[c, pl.ds(s * (DEG_R // NS), DEG_R // NS)])


_deg_kernel = functools.partial(
    pl.kernel,
    out_type=jax.ShapeDtypeStruct((NC, DEG_R), jnp.float32),
    mesh=_MESH,
    compiler_params=_SC_PARAMS,
    scratch_types=[
        pltpu.VMEM((MP_CH, MP_CS), jnp.int32),
        pltpu.VMEM((DEG_R,), jnp.float32),
        pltpu.VMEM((NS, DEG_R // NS), jnp.float32),
        pltpu.VMEM((DEG_R // NS,), jnp.float32),
        pltpu.VMEM_SHARED((NS, DEG_R), jnp.float32),
    ],
)(_deg_body)


def _mp_body(xs_hbm, psrc_hbm, pdst_hbm, cnt_hbm, zeros_hbm, out_hbm,
             idxs_v, idb0, idb1, idb2, idb3, r0, r1, r2, r3,
             sg0, sg1, sg2, sg3, si0, si1, si2, si3, cnt_v, acc_sh):
    c = lax.axis_index("c")
    s = lax.axis_index("s")
    w = c * NS + s
    pltpu.sync_copy(cnt_hbm.at[w], cnt_v)
    cnts = cnt_v[...]
    lanes = lax.iota(jnp.int32, 16)
    rows = (r0, r1, r2, r3)
    idbs = (idb0, idb1, idb2, idb3)
    sgs = (sg0, sg1, sg2, sg3)
    sis = (si0, si1, si2, si3)

    for p in range(MP_NPASS):
        lo = p * MP_RANGE
        base = (w * MP_NPASS + p) * P_CAP
        nquad = jnp.sum(jnp.where(lanes == p, cnts, 0))
        nchunk = nquad * 4
        pltpu.sync_copy(psrc_hbm.at[pl.ds(base, P_CAP)], idxs_v.at[pl.ds(0, P_CAP)])

        def ztail(i, carry):
            idxs_v[pl.ds(nchunk * MP_CS + i * 16, 16)] = jnp.zeros((16,), jnp.int32)
            return carry

        lax.fori_loop(0, P_TAIL // 16, ztail, 0)
        pltpu.sync_copy(zeros_hbm.at[pl.ds(0, ACC_ROWS // NS)],
                        acc_sh.at[pl.ds(s * (ACC_ROWS // NS), ACC_ROWS // NS)])
        plsc.subcore_barrier()

        def sidx(k):
            return idxs_v.at[pl.ds(k * MP_CS, MP_CS)]

        def g_issue(k, j):
            pltpu.async_copy(xs_hbm.at[sidx(k)], rows[j], sgs[j])

        def g_wait(k, j):
            pltpu.make_async_copy(xs_hbm.at[sidx(k)], rows[j], sgs[j]).wait()

        def i_issue(k, j):
            pltpu.async_copy(pdst_hbm.at[pl.ds(base + k * MP_CS, MP_CS)],
                             idbs[j], sis[j])

        def i_wait(k, j):
            pltpu.make_async_copy(pdst_hbm.at[pl.ds(base + k * MP_CS, MP_CS)],
                                  idbs[j], sis[j]).wait()

        for j in range(4):
            g_issue(j, j)
            i_issue(j, j)

        def quad(q, carry):
            for j in range(4):
                k = q * 4 + j
                g_wait(k, j)
                i_wait(k, j)
                pltpu.sync_copy(rows[j], acc_sh.at[idbs[j]], add=True)
                g_issue(k + 4, j)
                i_issue(k + 4, j)
            return carry

        lax.fori_loop(0, nquad, quad, 0)
        for j in range(4):
            g_wait(nchunk + j, j)
            i_wait(nchunk + j, j)

        plsc.subcore_barrier()

        @pl.when(s < 8)
        def _dump():
            pltpu.sync_copy(acc_sh.at[pl.ds(s * (MP_RANGE // 8), MP_RANGE // 8)],
                            out_hbm.at[c, pl.ds(lo + s * (MP_RANGE // 8), MP_RANGE // 8), :])
        plsc.subcore_barrier()


_mp_kernel = functools.partial(
    pl.kernel,
    out_type=jax.ShapeDtypeStruct((NC, NPOUT, FP), jnp.float32),
    mesh=_MESH,
    compiler_params=_SC_PARAMS,
    scratch_types=[
        pltpu.VMEM((P_CAP + P_TAIL,), jnp.int32),
        pltpu.VMEM((MP_CS,), jnp.int32),
        pltpu.VMEM((MP_CS,), jnp.int32),
        pltpu.VMEM((MP_CS,), jnp.int32),
        pltpu.VMEM((MP_CS,), jnp.int32),
        pltpu.VMEM((MP_CS, FP), jnp.float32),
        pltpu.VMEM((MP_CS, FP), jnp.float32),
        pltpu.VMEM((MP_CS, FP), jnp.float32),
        pltpu.VMEM((MP_CS, FP), jnp.float32),
        pltpu.SemaphoreType.DMA,
        pltpu.SemaphoreType.DMA,
        pltpu.SemaphoreType.DMA,
        pltpu.SemaphoreType.DMA,
        pltpu.SemaphoreType.DMA,
        pltpu.SemaphoreType.DMA,
        pltpu.SemaphoreType.DMA,
        pltpu.SemaphoreType.DMA,
        pltpu.VMEM((16,), jnp.int32),
        pltpu.VMEM_SHARED((ACC_ROWS, FP), jnp.float32),
    ],
)(_mp_body)


def _gru_body(x_ref, h0_ref, wih_ref, whh_ref, bih_ref, bhh_ref, out_ref):
    wih = wih_ref[...]
    whh = whh_ref[...]
    bih = bih_ref[...]
    bhh = bhh_ref[...]

    def step(t, h):
        xt = x_ref[t]
        gi = jnp.dot(xt, wih, preferred_element_type=jnp.float32) + bih
        gh = jnp.dot(h, whh, preferred_element_type=jnp.float32) + bhh
        r = jax.nn.sigmoid(gi[:, :HG] + gh[:, :HG])
        z = jax.nn.sigmoid(gi[:, HG:2 * HG] + gh[:, HG:2 * HG])
        n = jnp.tanh(gi[:, 2 * HG:] + r * gh[:, 2 * HG:])
        return (1.0 - z) * n + z * h

    out_ref[...] = lax.fori_loop(0, LSEQ, step, h0_ref[...])


def _gru_call(xseq, h0p, wihT, whhT, bih, bhh):
    return pl.pallas_call(
        _gru_body,
        grid=(NBLK,),
        in_specs=[
            pl.BlockSpec((LSEQ, BN, IP), lambda i: (0, i, 0)),
            pl.BlockSpec((BN, HG), lambda i: (i, 0)),
            pl.BlockSpec((IP, 3 * HG), lambda i: (0, 0)),
            pl.BlockSpec((HG, 3 * HG), lambda i: (0, 0)),
            pl.BlockSpec((1, 3 * HG), lambda i: (0, 0)),
            pl.BlockSpec((1, 3 * HG), lambda i: (0, 0)),
        ],
        out_specs=pl.BlockSpec((BN, HG), lambda i: (i, 0)),
        out_shape=jax.ShapeDtypeStruct((NP, HG), jnp.float32),
    )(xseq, h0p, wihT, whhT, bih, bhh)


def _lin1_body(ht_ref, w1_ref, dis_ref, out_ref):
    out_ref[...] = jnp.dot(ht_ref[...], w1_ref[...],
                           preferred_element_type=jnp.float32) * dis_ref[...]


def _lin1_call(hT, W1T, disc):
    return pl.pallas_call(
        _lin1_body,
        grid=(NBLK,),
        in_specs=[
            pl.BlockSpec((BN, HG), lambda i: (i, 0)),
            pl.BlockSpec((HG, FP), lambda i: (0, 0)),
            pl.BlockSpec((BN, 1), lambda i: (i, 0)),
        ],
        out_specs=pl.BlockSpec((BN, FP), lambda i: (i, 0)),
        out_shape=jax.ShapeDtypeStruct((NP, FP), jnp.float32),
    )(hT, W1T, disc)


def _onehot(idx_ref):
    return (idx_ref[...] == lax.broadcasted_iota(jnp.int32, (BN, HG), 1)
            ).astype(jnp.float32)


def _comb1_body(acc_ref, xs1_ref, dis_ref, htf_ref, idx_ref, b1_ref,
                w2a_ref, w2b_ref, g1_ref, xs2_ref):
    acc = acc_ref[...]
    dis = dis_ref[...]
    g1 = (acc[0] + acc[1] + xs1_ref[...]) * dis + b1_ref[...]
    g1_ref[...] = g1
    r1 = jnp.maximum(g1, 0.0)
    re = jnp.maximum(
        jnp.dot(_onehot(idx_ref), htf_ref[...], preferred_element_type=jnp.float32),
        0.0)
    xs2_ref[...] = (jnp.dot(r1, w2a_ref[...], preferred_element_type=jnp.float32)
                    + jnp.dot(re, w2b_ref[...], preferred_element_type=jnp.float32)
                    ) * dis


def _comb1_call(acc1, xs1, disc, hTf, idxcol, b1p, W2Ta, W2Tb):
    return pl.pallas_call(
        _comb1_body,
        grid=(NBLK,),
        in_specs=[
            pl.BlockSpec((NC, BN, FP), lambda i: (0, i, 0)),
            pl.BlockSpec((BN, FP), lambda i: (i, 0)),
            pl.BlockSpec((BN, 1), lambda i: (i, 0)),
            pl.BlockSpec((HG, HG), lambda i: (0, 0)),
            pl.BlockSpec((BN, 1), lambda i: (i, 0)),
            pl.BlockSpec((1, FP), lambda i: (0, 0)),
            pl.BlockSpec((FP, FP), lambda i: (0, 0)),
            pl.BlockSpec((HG, FP), lambda i: (0, 0)),
        ],
        out_specs=[
            pl.BlockSpec((BN, FP), lambda i: (i, 0)),
            pl.BlockSpec((BN, FP), lambda i: (i, 0)),
        ],
        out_shape=[
            jax.ShapeDtypeStruct((NP, FP), jnp.float32),
            jax.ShapeDtypeStruct((NP, FP), jnp.float32),
        ],
    )(acc1, xs1, disc, hTf, idxcol, b1p, W2Ta, W2Tb)


def _comb2_body(acc_ref, xs2_ref, dis_ref, g1f_ref, idx_ref, b2_ref, out_ref):
    acc = acc_ref[...]
    dis = dis_ref[...]
    o1 = jnp.maximum((acc[0] + acc[1] + xs2_ref[...]) * dis + b2_ref[...], 0.0)
    oh = (idx_ref[...] == lax.broadcasted_iota(jnp.int32, (BN7, HG), 1)
          ).astype(jnp.float32)
    o2 = jnp.dot(oh, g1f_ref[...], preferred_element_type=jnp.float32)
    out_ref[...] = jnp.concatenate([o1[:, :100], o2[:, :100]], axis=1)


def _comb2_call(acc2, xs2, disc, g1f, idxcol, b2p):
    return pl.pallas_call(
        _comb2_body,
        grid=(N // BN7,),
        in_specs=[
            pl.BlockSpec((NC, BN7, FP), lambda i: (0, i, 0)),
            pl.BlockSpec((BN7, FP), lambda i: (i, 0)),
            pl.BlockSpec((BN7, 1), lambda i: (i, 0)),
            pl.BlockSpec((HG, FP), lambda i: (0, 0)),
            pl.BlockSpec((BN7, 1), lambda i: (i, 0)),
            pl.BlockSpec((1, FP), lambda i: (0, 0)),
        ],
        out_specs=pl.BlockSpec((BN7, 200), lambda i: (i, 0)),
        out_shape=jax.ShapeDtypeStruct((N, 200), jnp.float32),
    )(acc2, xs2, disc, g1f, idxcol, b2p)


def kernel(merged_tree_feature, merged_tree_edge_index, indices, emb_table, h0,
           w_ih, w_hh, b_ih, b_hh, W1, b1, W2, b2):
    f32 = jnp.float32
    feat = merged_tree_feature.astype(jnp.int32)
    src2 = jnp.pad(merged_tree_edge_index[0].astype(jnp.int32),
                   (0, EP - E)).reshape(MP_ROWS, MP_CS)
    dst2 = jnp.pad(merged_tree_edge_index[1].astype(jnp.int32),
                   (0, EP - E), constant_values=N).reshape(MP_ROWS, MP_CS)
    idxcol = jnp.pad(indices.astype(jnp.int32), (0, NP - N)).reshape(NP, 1)

    idxflat = jnp.pad(feat.T, ((0, 0), (0, NP - N))).reshape(LSEQ * NP)
    embp = jnp.pad(emb_table.astype(f32), ((0, 0), (0, IP - 100)))

    h0p = jnp.pad(h0[0].astype(f32), ((0, NP - N), (0, 0)))
    wihT = jnp.pad(w_ih.T.astype(f32), ((0, IP - 100), (0, 0)))
    whhT = w_hh.T.astype(f32)
    bihr = b_ih.astype(f32).reshape(1, 3 * HG)
    bhhr = b_hh.astype(f32).reshape(1, 3 * HG)
    W1T = jnp.pad(W1.T.astype(f32), ((0, 0), (0, FP - 100)))
    b1p = jnp.pad(b1.astype(f32), (0, FP - 100)).reshape(1, FP)
    W2Ta = jnp.pad(W2[:, :100].T.astype(f32), ((0, FP - 100), (0, FP - 100)))
    W2Tb = jnp.pad(W2[:, 100:].T.astype(f32), ((0, 0), (0, FP - 100)))
    b2p = jnp.pad(b2.astype(f32), (0, FP - 100)).reshape(1, FP)

    zeros_fp = jnp.zeros((BN, FP), f32)
    ones_fp = jnp.ones((MP_CS, FP), f32)

    degp = _deg_kernel(dst2)
    disc = (1.0 / jnp.sqrt(degp[0, :NP] + degp[1, :NP] + 1.0)).reshape(NP, 1)
    xseq = _emb_kernel(idxflat, embp).reshape(LSEQ, NP, IP)
    hT = _gru_call(xseq, h0p, wihT, whhT, bihr, bhhr)
    psrc, pdst, pcnt = _part_kernel(src2, dst2)
    xs1 = _lin1_call(hT, W1T, disc)
    acc1 = _mp_kernel(xs1, psrc, pdst, pcnt, zeros_fp)
    g1, xs2 = _comb1_call(acc1, xs1, disc, hT[:HG], idxcol, b1p, W2Ta, W2Tb)
    acc2 = _mp_kernel(xs2, psrc, pdst, pcnt, zeros_fp)
    return _comb2_call(acc2, xs2, disc, g1[:HG], idxcol, b2p)


# speedup vs baseline: 1.8877x; 1.8877x over previous
import functools

import jax
import jax.numpy as jnp
from jax import lax
from jax.experimental import pallas as pl
from jax.experimental.pallas import tpu as pltpu
from jax.experimental.pallas import tpu_sc as plsc

N = 10000
E = 640000
LSEQ = 20
VOCAB = 100000
IP = 128
FP = 128
HG = 128
NP = 10112
BN = 632
NBLK = NP // BN
BN7 = 1000

NC = 2
NS = 16
NW = NC * NS

MP_NPASS = 3
P_BLK = 80
P_STG = 10752
P_CAP = 21504
P_DUMP = 10752
P_TAIL = 512
MP_RANGE = 3392
ACC_ROWS = 3456
JUNK = ACC_ROWS - 1
NPOUT = MP_NPASS * MP_RANGE
DEG_R = 10240

MP_CS = 128
MP_CH = 160
EP = NW * MP_CH * MP_CS
MP_ROWS = EP // MP_CS

EMB_CS = 80
EMB_CH = 79
EMB_PER_W = EMB_CH * EMB_CS

_MESH = plsc.VectorSubcoreMesh(core_axis_name="c", subcore_axis_name="s",
                               num_cores=NC, num_subcores=NS)
_SC_PARAMS = pltpu.CompilerParams(needs_layout_passes=False)


def _wid():
    return lax.axis_index("c") * NS + lax.axis_index("s")


def _emb_body(idx_hbm, table_hbm, out_hbm, idx_v, rows_a, rows_b, sem_a, sem_b):
    w = _wid()
    base = w * EMB_PER_W
    pltpu.sync_copy(idx_hbm.at[pl.ds(base, EMB_PER_W)], idx_v)

    def idx_at(k):
        return idx_v.at[pl.ds(k * EMB_CS, EMB_CS)]

    def gather(k, buf, sem):
        return pltpu.async_copy(table_hbm.at[idx_at(k)], buf, sem)

    def put(k, buf):
        pltpu.sync_copy(buf, out_hbm.at[pl.ds(base + k * EMB_CS, EMB_CS), :])

    gather(0, rows_a, sem_a)

    def pair(j, carry):
        gather(2 * j + 1, rows_b, sem_b)
        pltpu.make_async_copy(table_hbm.at[idx_at(2 * j)], rows_a, sem_a).wait()
        put(2 * j, rows_a)
        gather(2 * j + 2, rows_a, sem_a)
        pltpu.make_async_copy(table_hbm.at[idx_at(2 * j + 1)], rows_b, sem_b).wait()
        put(2 * j + 1, rows_b)
        return carry

    lax.fori_loop(0, (EMB_CH - 1) // 2, pair, 0)
    k_last = EMB_CH - 1
    pltpu.make_async_copy(table_hbm.at[idx_at(k_last)], rows_a, sem_a).wait()
    put(k_last, rows_a)


_emb_kernel = functools.partial(
    pl.kernel,
    out_type=jax.ShapeDtypeStruct((LSEQ * NP, IP), jnp.float32),
    mesh=_MESH,
    compiler_params=_SC_PARAMS,
    scratch_types=[
        pltpu.VMEM((EMB_PER_W,), jnp.int32),
        pltpu.VMEM((EMB_CS, IP), jnp.float32),
        pltpu.VMEM((EMB_CS, IP), jnp.float32),
        pltpu.SemaphoreType.DMA,
        pltpu.SemaphoreType.DMA,
    ],
)(_emb_body)


def _part_body(src2_hbm, dst2_hbm, psrc_hbm, pdst_hbm, cnt_hbm,
               sblk_v, dblk_v, st_s0, st_s1, st_s2, st_d0, st_d1, st_d2, cnt_v):
    c = lax.axis_index("c")
    s = lax.axis_index("s")
    w = c * NS + s
    st_s = (st_s0, st_s1, st_s2)
    st_d = (st_d0, st_d1, st_d2)
    goff = [jnp.int32(0)] * MP_NPASS

    for blk in range(2):
        pltpu.sync_copy(src2_hbm.at[pl.ds(w * MP_CH + blk * P_BLK, P_BLK)], sblk_v)
        pltpu.sync_copy(dst2_hbm.at[pl.ds(w * MP_CH + blk * P_BLK, P_BLK)], dblk_v)

        def sub(t, offs):
            k = t // (MP_CS // 16)
            i = t % (MP_CS // 16)
            vs = sblk_v[k, pl.ds(i * 16, 16)]
            vd = dblk_v[k, pl.ds(i * 16, 16)]
            new = []
            for p in range(MP_NPASS):
                loc = vd - p * MP_RANGE
                ok = (loc >= 0) & (loc < MP_RANGE)
                off = offs[p]
                ranks = plsc.cumsum(ok.astype(jnp.int32))
                pos = off + ranks - 1
                plsc.store_scatter(st_s[p], [pos], vs, mask=ok)
                plsc.store_scatter(st_d[p], [pos], loc, mask=ok)
                new.append(off + jnp.sum(ok.astype(jnp.int32)))
            return tuple(new)

        offs = lax.fori_loop(0, P_BLK * (MP_CS // 16), sub, (jnp.int32(0),) * MP_NPASS)

        lanes16 = lax.iota(jnp.int32, 16)
        zeros16v = jnp.zeros((16,), jnp.int32)
        junk16v = jnp.full((16,), JUNK, jnp.int32)
        for p in range(MP_NPASS):
            off = offs[p]
            if blk == 0:
                plsc.store_scatter(st_s[p], [off + lanes16], zeros16v)
                plsc.store_scatter(st_d[p], [off + lanes16], junk16v)
                off = (off + 15) & ~jnp.int32(15)
            else:
                for q in range(32):
                    plsc.store_scatter(st_s[p], [off + q * 16 + lanes16], zeros16v)
                    plsc.store_scatter(st_d[p], [off + q * 16 + lanes16], junk16v)
            base = pl.multiple_of((w * MP_NPASS + p) * P_CAP + goff[p], 16)
            pltpu.sync_copy(st_s[p].at[pl.ds(0, P_DUMP)], psrc_hbm.at[pl.ds(base, P_DUMP)])
            pltpu.sync_copy(st_d[p].at[pl.ds(0, P_DUMP)], pdst_hbm.at[pl.ds(base, P_DUMP)])
            goff[p] = goff[p] + off

    np0 = (jnp.maximum(goff[0], 1) + 511) // 512
    np1 = (jnp.maximum(goff[1], 1) + 511) // 512
    np2 = (jnp.maximum(goff[2], 1) + 511) // 512
    lanes = lax.iota(jnp.int32, 16)
    cnt_v[...] = (jnp.where(lanes == 0, np0, 0) + jnp.where(lanes == 1, np1, 0)
                  + jnp.where(lanes == 2, np2, 0))
    pltpu.sync_copy(cnt_v, cnt_hbm.at[w])


_part_kernel = functools.partial(
    pl.kernel,
    out_type=[
        jax.ShapeDtypeStruct((NW * MP_NPASS * P_CAP + P_TAIL,), jnp.int32),
        jax.ShapeDtypeStruct((NW * MP_NPASS * P_CAP + P_TAIL,), jnp.int32),
        jax.ShapeDtypeStruct((NW, 16), jnp.int32),
    ],
    mesh=_MESH,
    compiler_params=_SC_PARAMS,
    scratch_types=[
        pltpu.VMEM((P_BLK, MP_CS), jnp.int32),
        pltpu.VMEM((P_BLK, MP_CS), jnp.int32),
        pltpu.VMEM((P_STG,), jnp.int32),
        pltpu.VMEM((P_STG,), jnp.int32),
        pltpu.VMEM((P_STG,), jnp.int32),
        pltpu.VMEM((P_STG,), jnp.int32),
        pltpu.VMEM((P_STG,), jnp.int32),
        pltpu.VMEM((P_STG,), jnp.int32),
        pltpu.VMEM((16,), jnp.int32),
    ],
)(_part_body)


def _deg_body(dst2_hbm, out_hbm, idx_v, hist_v, buf_v, res_v, sh):
    c = lax.axis_index("c")
    s = lax.axis_index("s")
    w = c * NS + s
    pltpu.sync_copy(dst2_hbm.at[pl.ds(w * MP_CH, MP_CH)], idx_v)

    def zero(i, carry):
        hist_v[pl.ds(i * 16, 16)] = jnp.zeros((16,), jnp.float32)
        return carry

    lax.fori_loop(0, DEG_R // 16, zero, 0)

    def count(k, carry):
        for i in range(MP_CS // 16):
            d = idx_v[k, pl.ds(i * 16, 16)]
            cnt, last = plsc.scan_count(d)
            plsc.addupdate_scatter(hist_v, [d], cnt.astype(jnp.float32),
                                   mask=last)
        return carry

    lax.fori_loop(0, MP_CH, count, 0)

    pltpu.sync_copy(hist_v, sh.at[s])
    plsc.subcore_barrier()
    for r in range(NS):
        pltpu.sync_copy(sh.at[r, pl.ds(s * (DEG_R // NS), DEG_R // NS)],
                        buf_v.at[r])

    def red(j, carry):
        tot = buf_v[0, pl.ds(j * 16, 16)]
        for r in range(1, NS):
            tot = tot + buf_v[r, pl.ds(j * 16, 16)]
        res_v[pl.ds(j * 16, 16)] = tot
        return carry

    lax.fori_loop(0, DEG_R // NS // 16, red, 0)
    pltpu.sync_copy(res_v, out_hbm.at[c, pl.ds(s * (DEG_R // NS), DEG_R // NS)])


_deg_kernel = functools.partial(
    pl.kernel,
    out_type=jax.ShapeDtypeStruct((NC, DEG_R), jnp.float32),
    mesh=_MESH,
    compiler_params=_SC_PARAMS,
    scratch_types=[
        pltpu.VMEM((MP_CH, MP_CS), jnp.int32),
        pltpu.VMEM((DEG_R,), jnp.float32),
        pltpu.VMEM((NS, DEG_R // NS), jnp.float32),
        pltpu.VMEM((DEG_R // NS,), jnp.float32),
        pltpu.VMEM_SHARED((NS, DEG_R), jnp.float32),
    ],
)(_deg_body)


def _mp_body(xs_hbm, psrc_hbm, pdst_hbm, cnt_hbm, zeros_hbm, out_hbm,
             idxs_v, idb_a, idb_b, rows_a, rows_b,
             sem_a, sem_b, sem_ia, sem_ib, cnt_v, acc_sh):
    c = lax.axis_index("c")
    s = lax.axis_index("s")
    w = c * NS + s
    pltpu.sync_copy(cnt_hbm.at[w], cnt_v)
    cnts = cnt_v[...]
    lanes = lax.iota(jnp.int32, 16)

    for p in range(MP_NPASS):
        lo = p * MP_RANGE
        base = (w * MP_NPASS + p) * P_CAP
        npair = jnp.sum(jnp.where(lanes == p, cnts, 0)) * 2
        pltpu.sync_copy(psrc_hbm.at[pl.ds(base, P_CAP)], idxs_v)
        pltpu.sync_copy(zeros_hbm.at[pl.ds(0, ACC_ROWS // NS)],
                        acc_sh.at[pl.ds(s * (ACC_ROWS // NS), ACC_ROWS // NS)])
        plsc.subcore_barrier()

        def sidx(k):
            return idxs_v.at[pl.ds(k * MP_CS, MP_CS)]

        def g_issue(k, buf, sem):
            pltpu.async_copy(xs_hbm.at[sidx(k)], buf, sem)

        def g_wait(k, buf, sem):
            pltpu.make_async_copy(xs_hbm.at[sidx(k)], buf, sem).wait()

        def i_issue(k, ib, sem):
            pltpu.async_copy(pdst_hbm.at[pl.ds(base + k * MP_CS, MP_CS)], ib, sem)

        def i_wait(k, ib, sem):
            pltpu.make_async_copy(pdst_hbm.at[pl.ds(base + k * MP_CS, MP_CS)],
                                  ib, sem).wait()

        def scat(buf, ib):
            pltpu.sync_copy(buf, acc_sh.at[ib], add=True)

        g_issue(0, rows_a, sem_a)
        i_issue(0, idb_a, sem_ia)

        def pair(j, carry):
            g_issue(2 * j + 1, rows_b, sem_b)
            i_issue(2 * j + 1, idb_b, sem_ib)
            g_wait(2 * j, rows_a, sem_a)
            i_wait(2 * j, idb_a, sem_ia)
            scat(rows_a, idb_a)
            g_issue(2 * j + 2, rows_a, sem_a)
            i_issue(2 * j + 2, idb_a, sem_ia)
            g_wait(2 * j + 1, rows_b, sem_b)
            i_wait(2 * j + 1, idb_b, sem_ib)
            scat(rows_b, idb_b)
            return carry

        lax.fori_loop(0, npair - 1, pair, 0)
        k = 2 * npair - 2
        g_wait(k, rows_a, sem_a)
        i_wait(k, idb_a, sem_ia)
        scat(rows_a, idb_a)
        g_issue(k + 1, rows_b, sem_b)
        i_issue(k + 1, idb_b, sem_ib)
        g_wait(k + 1, rows_b, sem_b)
        i_wait(k + 1, idb_b, sem_ib)
        scat(rows_b, idb_b)

        plsc.subcore_barrier()

        @pl.when(s < 8)
        def _dump():
            pltpu.sync_copy(acc_sh.at[pl.ds(s * (MP_RANGE // 8), MP_RANGE // 8)],
                            out_hbm.at[c, pl.ds(lo + s * (MP_RANGE // 8), MP_RANGE // 8), :])
        plsc.subcore_barrier()


_mp_kernel = functools.partial(
    pl.kernel,
    out_type=jax.ShapeDtypeStruct((NC, NPOUT, FP), jnp.float32),
    mesh=_MESH,
    compiler_params=_SC_PARAMS,
    scratch_types=[
        pltpu.VMEM((P_CAP,), jnp.int32),
        pltpu.VMEM((MP_CS,), jnp.int32),
        pltpu.VMEM((MP_CS,), jnp.int32),
        pltpu.VMEM((MP_CS, FP), jnp.float32),
        pltpu.VMEM((MP_CS, FP), jnp.float32),
        pltpu.SemaphoreType.DMA,
        pltpu.SemaphoreType.DMA,
        pltpu.SemaphoreType.DMA,
        pltpu.SemaphoreType.DMA,
        pltpu.VMEM((16,), jnp.int32),
        pltpu.VMEM_SHARED((ACC_ROWS, FP), jnp.float32),
    ],
)(_mp_body)


def _gru_body(x_ref, h0_ref, wih_ref, whh_ref, bih_ref, bhh_ref, out_ref):
    wih = wih_ref[...]
    whh = whh_ref[...]
    bih = bih_ref[...]
    bhh = bhh_ref[...]

    def step(t, h):
        xt = x_ref[t]
        gi = jnp.dot(xt, wih, preferred_element_type=jnp.float32) + bih
        gh = jnp.dot(h, whh, preferred_element_type=jnp.float32) + bhh
        r = jax.nn.sigmoid(gi[:, :HG] + gh[:, :HG])
        z = jax.nn.sigmoid(gi[:, HG:2 * HG] + gh[:, HG:2 * HG])
        n = jnp.tanh(gi[:, 2 * HG:] + r * gh[:, 2 * HG:])
        return (1.0 - z) * n + z * h

    out_ref[...] = lax.fori_loop(0, LSEQ, step, h0_ref[...])


def _gru_call(xseq, h0p, wihT, whhT, bih, bhh):
    return pl.pallas_call(
        _gru_body,
        grid=(NBLK,),
        in_specs=[
            pl.BlockSpec((LSEQ, BN, IP), lambda i: (0, i, 0)),
            pl.BlockSpec((BN, HG), lambda i: (i, 0)),
            pl.BlockSpec((IP, 3 * HG), lambda i: (0, 0)),
            pl.BlockSpec((HG, 3 * HG), lambda i: (0, 0)),
            pl.BlockSpec((1, 3 * HG), lambda i: (0, 0)),
            pl.BlockSpec((1, 3 * HG), lambda i: (0, 0)),
        ],
        out_specs=pl.BlockSpec((BN, HG), lambda i: (i, 0)),
        out_shape=jax.ShapeDtypeStruct((NP, HG), jnp.float32),
    )(xseq, h0p, wihT, whhT, bih, bhh)


def _lin1_body(ht_ref, w1_ref, dis_ref, out_ref):
    out_ref[...] = jnp.dot(ht_ref[...], w1_ref[...],
                           preferred_element_type=jnp.float32) * dis_ref[...]


def _lin1_call(hT, W1T, disc):
    return pl.pallas_call(
        _lin1_body,
        grid=(NBLK,),
        in_specs=[
            pl.BlockSpec((BN, HG), lambda i: (i, 0)),
            pl.BlockSpec((HG, FP), lambda i: (0, 0)),
            pl.BlockSpec((BN, 1), lambda i: (i, 0)),
        ],
        out_specs=pl.BlockSpec((BN, FP), lambda i: (i, 0)),
        out_shape=jax.ShapeDtypeStruct((NP, FP), jnp.float32),
    )(hT, W1T, disc)


def _onehot(idx_ref):
    return (idx_ref[...] == lax.broadcasted_iota(jnp.int32, (BN, HG), 1)
            ).astype(jnp.float32)


def _comb1_body(acc_ref, xs1_ref, dis_ref, htf_ref, idx_ref, b1_ref,
                w2a_ref, w2b_ref, g1_ref, xs2_ref):
    acc = acc_ref[...]
    dis = dis_ref[...]
    g1 = (acc[0] + acc[1] + xs1_ref[...]) * dis + b1_ref[...]
    g1_ref[...] = g1
    r1 = jnp.maximum(g1, 0.0)
    re = jnp.maximum(
        jnp.dot(_onehot(idx_ref), htf_ref[...], preferred_element_type=jnp.float32),
        0.0)
    xs2_ref[...] = (jnp.dot(r1, w2a_ref[...], preferred_element_type=jnp.float32)
                    + jnp.dot(re, w2b_ref[...], preferred_element_type=jnp.float32)
                    ) * dis


def _comb1_call(acc1, xs1, disc, hTf, idxcol, b1p, W2Ta, W2Tb):
    return pl.pallas_call(
        _comb1_body,
        grid=(NBLK,),
        in_specs=[
            pl.BlockSpec((NC, BN, FP), lambda i: (0, i, 0)),
            pl.BlockSpec((BN, FP), lambda i: (i, 0)),
            pl.BlockSpec((BN, 1), lambda i: (i, 0)),
            pl.BlockSpec((HG, HG), lambda i: (0, 0)),
            pl.BlockSpec((BN, 1), lambda i: (i, 0)),
            pl.BlockSpec((1, FP), lambda i: (0, 0)),
            pl.BlockSpec((FP, FP), lambda i: (0, 0)),
            pl.BlockSpec((HG, FP), lambda i: (0, 0)),
        ],
        out_specs=[
            pl.BlockSpec((BN, FP), lambda i: (i, 0)),
            pl.BlockSpec((BN, FP), lambda i: (i, 0)),
        ],
        out_shape=[
            jax.ShapeDtypeStruct((NP, FP), jnp.float32),
            jax.ShapeDtypeStruct((NP, FP), jnp.float32),
        ],
    )(acc1, xs1, disc, hTf, idxcol, b1p, W2Ta, W2Tb)


def _comb2_body(acc_ref, xs2_ref, dis_ref, g1f_ref, idx_ref, b2_ref, out_ref):
    acc = acc_ref[...]
    dis = dis_ref[...]
    o1 = jnp.maximum((acc[0] + acc[1] + xs2_ref[...]) * dis + b2_ref[...], 0.0)
    oh = (idx_ref[...] == lax.broadcasted_iota(jnp.int32, (BN7, HG), 1)
          ).astype(jnp.float32)
    o2 = jnp.dot(oh, g1f_ref[...], preferred_element_type=jnp.float32)
    out_ref[...] = jnp.concatenate([o1[:, :100], o2[:, :100]], axis=1)


def _comb2_call(acc2, xs2, disc, g1f, idxcol, b2p):
    return pl.pallas_call(
        _comb2_body,
        grid=(N // BN7,),
        in_specs=[
            pl.BlockSpec((NC, BN7, FP), lambda i: (0, i, 0)),
            pl.BlockSpec((BN7, FP), lambda i: (i, 0)),
            pl.BlockSpec((BN7, 1), lambda i: (i, 0)),
            pl.BlockSpec((HG, FP), lambda i: (0, 0)),
            pl.BlockSpec((BN7, 1), lambda i: (i, 0)),
            pl.BlockSpec((1, FP), lambda i: (0, 0)),
        ],
        out_specs=pl.BlockSpec((BN7, 200), lambda i: (i, 0)),
        out_shape=jax.ShapeDtypeStruct((N, 200), jnp.float32),
    )(acc2, xs2, disc, g1f, idxcol, b2p)


def kernel(merged_tree_feature, merged_tree_edge_index, indices, emb_table, h0,
           w_ih, w_hh, b_ih, b_hh, W1, b1, W2, b2):
    f32 = jnp.float32
    feat = merged_tree_feature.astype(jnp.int32)
    src2 = jnp.pad(merged_tree_edge_index[0].astype(jnp.int32),
                   (0, EP - E)).reshape(MP_ROWS, MP_CS)
    dst2 = jnp.pad(merged_tree_edge_index[1].astype(jnp.int32),
                   (0, EP - E), constant_values=N).reshape(MP_ROWS, MP_CS)
    idxcol = jnp.pad(indices.astype(jnp.int32), (0, NP - N)).reshape(NP, 1)

    idxflat = jnp.pad(feat.T, ((0, 0), (0, NP - N))).reshape(LSEQ * NP)
    embp = jnp.pad(emb_table.astype(f32), ((0, 0), (0, IP - 100)))

    h0p = jnp.pad(h0[0].astype(f32), ((0, NP - N), (0, 0)))
    wihT = jnp.pad(w_ih.T.astype(f32), ((0, IP - 100), (0, 0)))
    whhT = w_hh.T.astype(f32)
    bihr = b_ih.astype(f32).reshape(1, 3 * HG)
    bhhr = b_hh.astype(f32).reshape(1, 3 * HG)
    W1T = jnp.pad(W1.T.astype(f32), ((0, 0), (0, FP - 100)))
    b1p = jnp.pad(b1.astype(f32), (0, FP - 100)).reshape(1, FP)
    W2Ta = jnp.pad(W2[:, :100].T.astype(f32), ((0, FP - 100), (0, FP - 100)))
    W2Tb = jnp.pad(W2[:, 100:].T.astype(f32), ((0, 0), (0, FP - 100)))
    b2p = jnp.pad(b2.astype(f32), (0, FP - 100)).reshape(1, FP)

    zeros_fp = jnp.zeros((BN, FP), f32)
    ones_fp = jnp.ones((MP_CS, FP), f32)

    degp = _deg_kernel(dst2)
    disc = (1.0 / jnp.sqrt(degp[0, :NP] + degp[1, :NP] + 1.0)).reshape(NP, 1)
    xseq = _emb_kernel(idxflat, embp).reshape(LSEQ, NP, IP)
    hT = _gru_call(xseq, h0p, wihT, whhT, bihr, bhhr)
    psrc, pdst, pcnt = _part_kernel(src2, dst2)
    xs1 = _lin1_call(hT, W1T, disc)
    acc1 = _mp_kernel(xs1, psrc, pdst, pcnt, zeros_fp)
    g1, xs2 = _comb1_call(acc1, xs1, disc, hT[:HG], idxcol, b1p, W2Ta, W2Tb)
    acc2 = _mp_kernel(xs2, psrc, pdst, pcnt, zeros_fp)
    return _comb2_call(acc2, xs2, disc, g1[:HG], idxcol, b2p)


# speedup vs baseline: 2.1285x; 1.1275x over previous
import functools

import jax
import jax.numpy as jnp
from jax import lax
from jax.experimental import pallas as pl
from jax.experimental.pallas import tpu as pltpu
from jax.experimental.pallas import tpu_sc as plsc

N = 10000
E = 640000
LSEQ = 20
VOCAB = 100000
IP = 128
FP = 128
HG = 128
NP = 10112
BN = 632
NBLK = NP // BN
BN7 = 1000

NC = 2
NS = 16
NW = NC * NS

MP_NPASS = 3
P_BLK = 80
P_STG = 10752
P_CAP = 21504
P_DUMP = 10752
P_TAIL = 512
MP_RANGE = 3392
ACC_ROWS = 3456
JUNK = ACC_ROWS - 1
NPOUT = MP_NPASS * MP_RANGE
DEG_R = 10240

MP_CS = 128
MP_CH = 160
EP = NW * MP_CH * MP_CS
MP_ROWS = EP // MP_CS

EMB_CS = 80
EMB_CH = 79
EMB_PER_W = EMB_CH * EMB_CS

_MESH = plsc.VectorSubcoreMesh(core_axis_name="c", subcore_axis_name="s",
                               num_cores=NC, num_subcores=NS)
_SC_PARAMS = pltpu.CompilerParams(needs_layout_passes=False)


def _wid():
    return lax.axis_index("c") * NS + lax.axis_index("s")


def _emb_body(idx_hbm, table_hbm, out_hbm, idx_v, rows_a, rows_b, sem_a, sem_b):
    w = _wid()
    base = w * EMB_PER_W
    pltpu.sync_copy(idx_hbm.at[pl.ds(base, EMB_PER_W)], idx_v)

    def idx_at(k):
        return idx_v.at[pl.ds(k * EMB_CS, EMB_CS)]

    def gather(k, buf, sem):
        return pltpu.async_copy(table_hbm.at[idx_at(k)], buf, sem)

    def put(k, buf):
        pltpu.sync_copy(buf, out_hbm.at[pl.ds(base + k * EMB_CS, EMB_CS), :])

    gather(0, rows_a, sem_a)

    def pair(j, carry):
        gather(2 * j + 1, rows_b, sem_b)
        pltpu.make_async_copy(table_hbm.at[idx_at(2 * j)], rows_a, sem_a).wait()
        put(2 * j, rows_a)
        gather(2 * j + 2, rows_a, sem_a)
        pltpu.make_async_copy(table_hbm.at[idx_at(2 * j + 1)], rows_b, sem_b).wait()
        put(2 * j + 1, rows_b)
        return carry

    lax.fori_loop(0, (EMB_CH - 1) // 2, pair, 0)
    k_last = EMB_CH - 1
    pltpu.make_async_copy(table_hbm.at[idx_at(k_last)], rows_a, sem_a).wait()
    put(k_last, rows_a)


_emb_kernel = functools.partial(
    pl.kernel,
    out_type=jax.ShapeDtypeStruct((LSEQ * NP, IP), jnp.float32),
    mesh=_MESH,
    compiler_params=_SC_PARAMS,
    scratch_types=[
        pltpu.VMEM((EMB_PER_W,), jnp.int32),
        pltpu.VMEM((EMB_CS, IP), jnp.float32),
        pltpu.VMEM((EMB_CS, IP), jnp.float32),
        pltpu.SemaphoreType.DMA,
        pltpu.SemaphoreType.DMA,
    ],
)(_emb_body)


def _part_body(src2_hbm, dst2_hbm, psrc_hbm, pdst_hbm, cnt_hbm,
               sblk_v, dblk_v, st_s0, st_s1, st_s2, st_d0, st_d1, st_d2, cnt_v):
    c = lax.axis_index("c")
    s = lax.axis_index("s")
    w = c * NS + s
    st_s = (st_s0, st_s1, st_s2)
    st_d = (st_d0, st_d1, st_d2)
    goff = [jnp.int32(0)] * MP_NPASS

    for blk in range(2):
        pltpu.sync_copy(src2_hbm.at[pl.ds(w * MP_CH + blk * P_BLK, P_BLK)], sblk_v)
        pltpu.sync_copy(dst2_hbm.at[pl.ds(w * MP_CH + blk * P_BLK, P_BLK)], dblk_v)

        def sub(t, offs):
            k = t // (MP_CS // 16)
            i = t % (MP_CS // 16)
            vs = sblk_v[k, pl.ds(i * 16, 16)]
            vd = dblk_v[k, pl.ds(i * 16, 16)]
            new = []
            for p in range(MP_NPASS):
                loc = vd - p * MP_RANGE
                ok = (loc >= 0) & (loc < MP_RANGE)
                off = offs[p]
                ranks = plsc.cumsum(ok.astype(jnp.int32))
                pos = off + ranks - 1
                plsc.store_scatter(st_s[p], [pos], vs, mask=ok)
                plsc.store_scatter(st_d[p], [pos], loc, mask=ok)
                new.append(off + jnp.sum(ok.astype(jnp.int32)))
            return tuple(new)

        offs = lax.fori_loop(0, P_BLK * (MP_CS // 16), sub, (jnp.int32(0),) * MP_NPASS)

        lanes16 = lax.iota(jnp.int32, 16)
        zeros16v = jnp.zeros((16,), jnp.int32)
        junk16v = jnp.full((16,), JUNK, jnp.int32)
        for p in range(MP_NPASS):
            off = offs[p]
            if blk == 0:
                plsc.store_scatter(st_s[p], [off + lanes16], zeros16v)
                plsc.store_scatter(st_d[p], [off + lanes16], junk16v)
                off = (off + 15) & ~jnp.int32(15)
            else:
                for q in range(16):
                    plsc.store_scatter(st_s[p], [off + q * 16 + lanes16], zeros16v)
                    plsc.store_scatter(st_d[p], [off + q * 16 + lanes16], junk16v)
            base = pl.multiple_of((w * MP_NPASS + p) * P_CAP + goff[p], 16)
            pltpu.sync_copy(st_s[p].at[pl.ds(0, P_DUMP)], psrc_hbm.at[pl.ds(base, P_DUMP)])
            pltpu.sync_copy(st_d[p].at[pl.ds(0, P_DUMP)], pdst_hbm.at[pl.ds(base, P_DUMP)])
            goff[p] = goff[p] + off

    np0 = (jnp.maximum(goff[0], 1) + 255) // 256
    np1 = (jnp.maximum(goff[1], 1) + 255) // 256
    np2 = (jnp.maximum(goff[2], 1) + 255) // 256
    lanes = lax.iota(jnp.int32, 16)
    cnt_v[...] = (jnp.where(lanes == 0, np0, 0) + jnp.where(lanes == 1, np1, 0)
                  + jnp.where(lanes == 2, np2, 0))
    pltpu.sync_copy(cnt_v, cnt_hbm.at[w])


_part_kernel = functools.partial(
    pl.kernel,
    out_type=[
        jax.ShapeDtypeStruct((NW * MP_NPASS * P_CAP + P_TAIL,), jnp.int32),
        jax.ShapeDtypeStruct((NW * MP_NPASS * P_CAP + P_TAIL,), jnp.int32),
        jax.ShapeDtypeStruct((NW, 16), jnp.int32),
    ],
    mesh=_MESH,
    compiler_params=_SC_PARAMS,
    scratch_types=[
        pltpu.VMEM((P_BLK, MP_CS), jnp.int32),
        pltpu.VMEM((P_BLK, MP_CS), jnp.int32),
        pltpu.VMEM((P_STG,), jnp.int32),
        pltpu.VMEM((P_STG,), jnp.int32),
        pltpu.VMEM((P_STG,), jnp.int32),
        pltpu.VMEM((P_STG,), jnp.int32),
        pltpu.VMEM((P_STG,), jnp.int32),
        pltpu.VMEM((P_STG,), jnp.int32),
        pltpu.VMEM((16,), jnp.int32),
    ],
)(_part_body)


def _deg_body(dst2_hbm, out_hbm, idx_v, hist_v, buf_v, res_v, sh):
    c = lax.axis_index("c")
    s = lax.axis_index("s")
    w = c * NS + s
    pltpu.sync_copy(dst2_hbm.at[pl.ds(w * MP_CH, MP_CH)], idx_v)

    def zero(i, carry):
        hist_v[pl.ds(i * 16, 16)] = jnp.zeros((16,), jnp.float32)
        return carry

    lax.fori_loop(0, DEG_R // 16, zero, 0)

    def count(k, carry):
        for i in range(MP_CS // 16):
            d = idx_v[k, pl.ds(i * 16, 16)]
            cnt, last = plsc.scan_count(d)
            plsc.addupdate_scatter(hist_v, [d], cnt.astype(jnp.float32),
                                   mask=last)
        return carry

    lax.fori_loop(0, MP_CH, count, 0)

    pltpu.sync_copy(hist_v, sh.at[s])
    plsc.subcore_barrier()
    for r in range(NS):
        pltpu.sync_copy(sh.at[r, pl.ds(s * (DEG_R // NS), DEG_R // NS)],
                        buf_v.at[r])

    def red(j, carry):
        tot = buf_v[0, pl.ds(j * 16, 16)]
        for r in range(1, NS):
            tot = tot + buf_v[r, pl.ds(j * 16, 16)]
        res_v[pl.ds(j * 16, 16)] = tot
        return carry

    lax.fori_loop(0, DEG_R // NS // 16, red, 0)
    pltpu.sync_copy(res_v, out_hbm.at[c, pl.ds(s * (DEG_R // NS), DEG_R // NS)])


_deg_kernel = functools.partial(
    pl.kernel,
    out_type=jax.ShapeDtypeStruct((NC, DEG_R), jnp.float32),
    mesh=_MESH,
    compiler_params=_SC_PARAMS,
    scratch_types=[
        pltpu.VMEM((MP_CH, MP_CS), jnp.int32),
        pltpu.VMEM((DEG_R,), jnp.float32),
        pltpu.VMEM((NS, DEG_R // NS), jnp.float32),
        pltpu.VMEM((DEG_R // NS,), jnp.float32),
        pltpu.VMEM_SHARED((NS, DEG_R), jnp.float32),
    ],
)(_deg_body)


def _mp_body(xs_hbm, psrc_hbm, pdst_hbm, cnt_hbm, zeros_hbm, out_hbm,
             idxs_v, idb_a, idb_b, rows_a, rows_b,
             sem_a, sem_b, sem_ia, sem_ib, cnt_v, acc_sh):
    c = lax.axis_index("c")
    s = lax.axis_index("s")
    w = c * NS + s
    pltpu.sync_copy(cnt_hbm.at[w], cnt_v)
    cnts = cnt_v[...]
    lanes = lax.iota(jnp.int32, 16)

    for p in range(MP_NPASS):
        lo = p * MP_RANGE
        base = (w * MP_NPASS + p) * P_CAP
        npair = jnp.sum(jnp.where(lanes == p, cnts, 0))
        pltpu.sync_copy(psrc_hbm.at[pl.ds(base, P_CAP)], idxs_v)
        pltpu.sync_copy(zeros_hbm.at[pl.ds(0, ACC_ROWS // NS)],
                        acc_sh.at[pl.ds(s * (ACC_ROWS // NS), ACC_ROWS // NS)])
        plsc.subcore_barrier()

        def sidx(k):
            return idxs_v.at[pl.ds(k * MP_CS, MP_CS)]

        def g_issue(k, buf, sem):
            pltpu.async_copy(xs_hbm.at[sidx(k)], buf, sem)

        def g_wait(k, buf, sem):
            pltpu.make_async_copy(xs_hbm.at[sidx(k)], buf, sem).wait()

        def i_issue(k, ib, sem):
            pltpu.async_copy(pdst_hbm.at[pl.ds(base + k * MP_CS, MP_CS)], ib, sem)

        def i_wait(k, ib, sem):
            pltpu.make_async_copy(pdst_hbm.at[pl.ds(base + k * MP_CS, MP_CS)],
                                  ib, sem).wait()

        def scat(buf, ib):
            pltpu.sync_copy(buf, acc_sh.at[ib], add=True)

        g_issue(0, rows_a, sem_a)
        i_issue(0, idb_a, sem_ia)

        def pair(j, carry):
            g_issue(2 * j + 1, rows_b, sem_b)
            i_issue(2 * j + 1, idb_b, sem_ib)
            g_wait(2 * j, rows_a, sem_a)
            i_wait(2 * j, idb_a, sem_ia)
            scat(rows_a, idb_a)
            g_issue(2 * j + 2, rows_a, sem_a)
            i_issue(2 * j + 2, idb_a, sem_ia)
            g_wait(2 * j + 1, rows_b, sem_b)
            i_wait(2 * j + 1, idb_b, sem_ib)
            scat(rows_b, idb_b)
            return carry

        lax.fori_loop(0, npair - 1, pair, 0)
        k = 2 * npair - 2
        g_wait(k, rows_a, sem_a)
        i_wait(k, idb_a, sem_ia)
        scat(rows_a, idb_a)
        g_issue(k + 1, rows_b, sem_b)
        i_issue(k + 1, idb_b, sem_ib)
        g_wait(k + 1, rows_b, sem_b)
        i_wait(k + 1, idb_b, sem_ib)
        scat(rows_b, idb_b)

        plsc.subcore_barrier()

        @pl.when(s < 8)
        def _dump():
            pltpu.sync_copy(acc_sh.at[pl.ds(s * (MP_RANGE // 8), MP_RANGE // 8)],
                            out_hbm.at[c, pl.ds(lo + s * (MP_RANGE // 8), MP_RANGE // 8), :])
        plsc.subcore_barrier()


_mp_kernel = functools.partial(
    pl.kernel,
    out_type=jax.ShapeDtypeStruct((NC, NPOUT, FP), jnp.float32),
    mesh=_MESH,
    compiler_params=_SC_PARAMS,
    scratch_types=[
        pltpu.VMEM((P_CAP,), jnp.int32),
        pltpu.VMEM((MP_CS,), jnp.int32),
        pltpu.VMEM((MP_CS,), jnp.int32),
        pltpu.VMEM((MP_CS, FP), jnp.float32),
        pltpu.VMEM((MP_CS, FP), jnp.float32),
        pltpu.SemaphoreType.DMA,
        pltpu.SemaphoreType.DMA,
        pltpu.SemaphoreType.DMA,
        pltpu.SemaphoreType.DMA,
        pltpu.VMEM((16,), jnp.int32),
        pltpu.VMEM_SHARED((ACC_ROWS, FP), jnp.float32),
    ],
)(_mp_body)


def _gru_body(x_ref, h0_ref, wih_ref, whh_ref, bih_ref, bhh_ref, out_ref):
    wih = wih_ref[...]
    whh = whh_ref[...]
    bih = bih_ref[...]
    bhh = bhh_ref[...]

    def step(t, h):
        xt = x_ref[t]
        gi = jnp.dot(xt, wih, preferred_element_type=jnp.float32) + bih
        gh = jnp.dot(h, whh, preferred_element_type=jnp.float32) + bhh
        r = jax.nn.sigmoid(gi[:, :HG] + gh[:, :HG])
        z = jax.nn.sigmoid(gi[:, HG:2 * HG] + gh[:, HG:2 * HG])
        n = jnp.tanh(gi[:, 2 * HG:] + r * gh[:, 2 * HG:])
        return (1.0 - z) * n + z * h

    out_ref[...] = lax.fori_loop(0, LSEQ, step, h0_ref[...])


def _gru_call(xseq, h0p, wihT, whhT, bih, bhh):
    return pl.pallas_call(
        _gru_body,
        grid=(NBLK,),
        in_specs=[
            pl.BlockSpec((LSEQ, BN, IP), lambda i: (0, i, 0)),
            pl.BlockSpec((BN, HG), lambda i: (i, 0)),
            pl.BlockSpec((IP, 3 * HG), lambda i: (0, 0)),
            pl.BlockSpec((HG, 3 * HG), lambda i: (0, 0)),
            pl.BlockSpec((1, 3 * HG), lambda i: (0, 0)),
            pl.BlockSpec((1, 3 * HG), lambda i: (0, 0)),
        ],
        out_specs=pl.BlockSpec((BN, HG), lambda i: (i, 0)),
        out_shape=jax.ShapeDtypeStruct((NP, HG), jnp.float32),
    )(xseq, h0p, wihT, whhT, bih, bhh)


def _lin1_body(ht_ref, w1_ref, dis_ref, out_ref):
    out_ref[...] = jnp.dot(ht_ref[...], w1_ref[...],
                           preferred_element_type=jnp.float32) * dis_ref[...]


def _lin1_call(hT, W1T, disc):
    return pl.pallas_call(
        _lin1_body,
        grid=(NBLK,),
        in_specs=[
            pl.BlockSpec((BN, HG), lambda i: (i, 0)),
            pl.BlockSpec((HG, FP), lambda i: (0, 0)),
            pl.BlockSpec((BN, 1), lambda i: (i, 0)),
        ],
        out_specs=pl.BlockSpec((BN, FP), lambda i: (i, 0)),
        out_shape=jax.ShapeDtypeStruct((NP, FP), jnp.float32),
    )(hT, W1T, disc)


def _onehot(idx_ref):
    return (idx_ref[...] == lax.broadcasted_iota(jnp.int32, (BN, HG), 1)
            ).astype(jnp.float32)


def _comb1_body(acc_ref, xs1_ref, dis_ref, htf_ref, idx_ref, b1_ref,
                w2a_ref, w2b_ref, g1_ref, xs2_ref):
    acc = acc_ref[...]
    dis = dis_ref[...]
    g1 = (acc[0] + acc[1] + xs1_ref[...]) * dis + b1_ref[...]
    g1_ref[...] = g1
    r1 = jnp.maximum(g1, 0.0)
    re = jnp.maximum(
        jnp.dot(_onehot(idx_ref), htf_ref[...], preferred_element_type=jnp.float32),
        0.0)
    xs2_ref[...] = (jnp.dot(r1, w2a_ref[...], preferred_element_type=jnp.float32)
                    + jnp.dot(re, w2b_ref[...], preferred_element_type=jnp.float32)
                    ) * dis


def _comb1_call(acc1, xs1, disc, hTf, idxcol, b1p, W2Ta, W2Tb):
    return pl.pallas_call(
        _comb1_body,
        grid=(NBLK,),
        in_specs=[
            pl.BlockSpec((NC, BN, FP), lambda i: (0, i, 0)),
            pl.BlockSpec((BN, FP), lambda i: (i, 0)),
            pl.BlockSpec((BN, 1), lambda i: (i, 0)),
            pl.BlockSpec((HG, HG), lambda i: (0, 0)),
            pl.BlockSpec((BN, 1), lambda i: (i, 0)),
            pl.BlockSpec((1, FP), lambda i: (0, 0)),
            pl.BlockSpec((FP, FP), lambda i: (0, 0)),
            pl.BlockSpec((HG, FP), lambda i: (0, 0)),
        ],
        out_specs=[
            pl.BlockSpec((BN, FP), lambda i: (i, 0)),
            pl.BlockSpec((BN, FP), lambda i: (i, 0)),
        ],
        out_shape=[
            jax.ShapeDtypeStruct((NP, FP), jnp.float32),
            jax.ShapeDtypeStruct((NP, FP), jnp.float32),
        ],
    )(acc1, xs1, disc, hTf, idxcol, b1p, W2Ta, W2Tb)


def _comb2_body(acc_ref, xs2_ref, dis_ref, g1f_ref, idx_ref, b2_ref, out_ref):
    acc = acc_ref[...]
    dis = dis_ref[...]
    o1 = jnp.maximum((acc[0] + acc[1] + xs2_ref[...]) * dis + b2_ref[...], 0.0)
    oh = (idx_ref[...] == lax.broadcasted_iota(jnp.int32, (BN7, HG), 1)
          ).astype(jnp.float32)
    o2 = jnp.dot(oh, g1f_ref[...], preferred_element_type=jnp.float32)
    out_ref[...] = jnp.concatenate([o1[:, :100], o2[:, :100]], axis=1)


def _comb2_call(acc2, xs2, disc, g1f, idxcol, b2p):
    return pl.pallas_call(
        _comb2_body,
        grid=(N // BN7,),
        in_specs=[
            pl.BlockSpec((NC, BN7, FP), lambda i: (0, i, 0)),
            pl.BlockSpec((BN7, FP), lambda i: (i, 0)),
            pl.BlockSpec((BN7, 1), lambda i: (i, 0)),
            pl.BlockSpec((HG, FP), lambda i: (0, 0)),
            pl.BlockSpec((BN7, 1), lambda i: (i, 0)),
            pl.BlockSpec((1, FP), lambda i: (0, 0)),
        ],
        out_specs=pl.BlockSpec((BN7, 200), lambda i: (i, 0)),
        out_shape=jax.ShapeDtypeStruct((N, 200), jnp.float32),
    )(acc2, xs2, disc, g1f, idxcol, b2p)


def kernel(merged_tree_feature, merged_tree_edge_index, indices, emb_table, h0,
           w_ih, w_hh, b_ih, b_hh, W1, b1, W2, b2):
    f32 = jnp.float32
    feat = merged_tree_feature.astype(jnp.int32)
    src2 = jnp.pad(merged_tree_edge_index[0].astype(jnp.int32),
                   (0, EP - E)).reshape(MP_ROWS, MP_CS)
    dst2 = jnp.pad(merged_tree_edge_index[1].astype(jnp.int32),
                   (0, EP - E), constant_values=N).reshape(MP_ROWS, MP_CS)
    idxcol = jnp.pad(indices.astype(jnp.int32), (0, NP - N)).reshape(NP, 1)

    idxflat = jnp.pad(feat.T, ((0, 0), (0, NP - N))).reshape(LSEQ * NP)
    embp = jnp.pad(emb_table.astype(f32), ((0, 0), (0, IP - 100)))

    h0p = jnp.pad(h0[0].astype(f32), ((0, NP - N), (0, 0)))
    wihT = jnp.pad(w_ih.T.astype(f32), ((0, IP - 100), (0, 0)))
    whhT = w_hh.T.astype(f32)
    bihr = b_ih.astype(f32).reshape(1, 3 * HG)
    bhhr = b_hh.astype(f32).reshape(1, 3 * HG)
    W1T = jnp.pad(W1.T.astype(f32), ((0, 0), (0, FP - 100)))
    b1p = jnp.pad(b1.astype(f32), (0, FP - 100)).reshape(1, FP)
    W2Ta = jnp.pad(W2[:, :100].T.astype(f32), ((0, FP - 100), (0, FP - 100)))
    W2Tb = jnp.pad(W2[:, 100:].T.astype(f32), ((0, 0), (0, FP - 100)))
    b2p = jnp.pad(b2.astype(f32), (0, FP - 100)).reshape(1, FP)

    zeros_fp = jnp.zeros((BN, FP), f32)
    ones_fp = jnp.ones((MP_CS, FP), f32)

    degp = _deg_kernel(dst2)
    disc = (1.0 / jnp.sqrt(degp[0, :NP] + degp[1, :NP] + 1.0)).reshape(NP, 1)
    xseq = _emb_kernel(idxflat, embp).reshape(LSEQ, NP, IP)
    hT = _gru_call(xseq, h0p, wihT, whhT, bihr, bhhr)
    psrc, pdst, pcnt = _part_kernel(src2, dst2)
    xs1 = _lin1_call(hT, W1T, disc)
    acc1 = _mp_kernel(xs1, psrc, pdst, pcnt, zeros_fp)
    g1, xs2 = _comb1_call(acc1, xs1, disc, hT[:HG], idxcol, b1p, W2Ta, W2Tb)
    acc2 = _mp_kernel(xs2, psrc, pdst, pcnt, zeros_fp)
    return _comb2_call(acc2, xs2, disc, g1[:HG], idxcol, b2p)
